# baseline (speedup 1.0000x reference)
import functools

import jax
import jax.numpy as jnp
from jax import lax
from jax.experimental import pallas as pl
from jax.experimental.pallas import tpu as pltpu
from jax.experimental.pallas import tpu_sc as plsc

DEPTH = 8192
D = 1024
T = 4096
B = 4

NC = 2
NS = 16
NW = NC * NS
ROWS = T // NW
CH = 16
NCHUNK = ROWS // CH
NBUF = 4
HCH = 8
NHR = ROWS // HCH
HBUF = 4

_MESH = plsc.VectorSubcoreMesh(core_axis_name="c", subcore_axis_name="s")


@functools.partial(
    pl.kernel,
    mesh=_MESH,
    out_type=(
        jax.ShapeDtypeStruct((DEPTH, D), jnp.float32),
        jax.ShapeDtypeStruct((DEPTH,), jnp.float32),
    ),
    scratch_types=[
        pltpu.VMEM((NBUF, CH, D), jnp.float32),
        pltpu.VMEM((NCHUNK, B, 16), jnp.int32),
        pltpu.VMEM_SHARED((NS, HBUF, HCH, D), jnp.float32),
        pltpu.VMEM((16,), jnp.float32),
        pltpu.VMEM((T,), jnp.float32),
        [pltpu.SemaphoreType.DMA] * NBUF,
        [pltpu.SemaphoreType.DMA] * NBUF,
        [pltpu.SemaphoreType.DMA] * NBUF,
        [pltpu.SemaphoreType.DMA] * HBUF,
        pltpu.SemaphoreType.DMA,
    ],
)
def _sc_kernel(x_hbm, off_hbm, hist_hbm, tin_hbm, out_hbm, tout_hbm,
               obuf, idxall, hshared, offbuf, tbuf,
               g0sems, gasems, osems, hsems, tsem):
    cid = lax.axis_index("c")
    sid = lax.axis_index("s")
    wid = sid * NC + cid
    base = wid * ROWS

    @pl.when(wid == NW - 1)
    def _():
        pltpu.async_copy(tin_hbm.at[pl.ds(T, T)], tbuf, tsem)

    @pl.when(wid == NW - 2)
    def _():
        pltpu.sync_copy(off_hbm, offbuf)
        offv = offbuf[...]

        @plsc.parallel_loop(0, T // 16, 1, unroll=4)
        def _(j):
            tbuf[pl.ds(j * 16, 16)] = offv + lax.convert_element_type(j * 16, jnp.float32)

        pltpu.async_copy(tbuf, tout_hbm.at[pl.ds(T, T)], tsem)

    iota = lax.iota(jnp.int32, 16)

    @plsc.parallel_loop(0, NCHUNK * B, 1, unroll=2)
    def _(i):
        c = i // B
        g = i % B
        idxall[c, g, pl.ds(0, 16)] = iota * B + ((base + c * CH) * B + g)

    def h_in(r):
        return pltpu.make_async_copy(
            hist_hbm.at[pl.ds(T + base + r * HCH, HCH)],
            hshared.at[sid, r % HBUF], hsems[r % HBUF])

    def h_out(r):
        return pltpu.make_async_copy(
            hshared.at[sid, r % HBUF],
            out_hbm.at[pl.ds(base + r * HCH, HCH)], hsems[r % HBUF])

    for r in range(HBUF):
        h_in(r).start()

    def g_start(c, g):
        b = c % NBUF
        sem = g0sems[b] if g == 0 else gasems[b]
        pltpu.async_copy(x_hbm.at[idxall.at[c, g]], obuf.at[b], sem, add=(g > 0))

    def g_wait(c, g):
        b = c % NBUF
        sem = g0sems[b] if g == 0 else gasems[b]
        pltpu.make_async_copy(x_hbm.at[idxall.at[c, g]], obuf.at[b], sem).wait()

    def o_copy(c):
        b = c % NBUF
        return pltpu.make_async_copy(
            obuf.at[b], out_hbm.at[pl.ds(T + base + c * CH, CH)], osems[b])

    g_start(0, 0)
    g_start(1, 0)
    g_wait(0, 0)
    for g in range(1, B):
        g_start(0, g)

    for c in range(NCHUNK):
        b = c % NBUF

        if c + 1 < NCHUNK:
            g_wait(c + 1, 0)
            for g in range(1, B):
                g_start(c + 1, g)
        if c + 2 < NCHUNK:
            if c - 2 >= 0:
                o_copy(c - 2).wait()
            g_start(c + 2, 0)

        for r in (2 * (c - 1), 2 * (c - 1) + 1):
            if r >= 0 and r + HBUF < NHR:
                h_out(r).wait()
                h_in(r + HBUF).start()
        for r in (2 * c, 2 * c + 1):
            h_in(r).wait()
            h_out(r).start()

        for g in range(1, B):
            g_wait(c, g)
        ob = obuf.at[b]

        @plsc.parallel_loop(0, CH * (D // 16), 1, unroll=4)
        def _(i, ob=ob):
            r = i // (D // 16)
            jo = (i % (D // 16)) * 16
            ob[r, pl.ds(jo, 16)] = ob[r, pl.ds(jo, 16)] * 0.25

        o_copy(c).start()

    for c in range(NCHUNK - 2, NCHUNK):
        o_copy(c).wait()
    for r in (NHR - 4, NHR - 3, NHR - 2, NHR - 1):
        h_out(r).wait()

    @pl.when(wid == NW - 1)
    def _():
        pltpu.make_async_copy(tin_hbm.at[pl.ds(T, T)], tbuf, tsem).wait()
        pltpu.sync_copy(tbuf, tout_hbm.at[pl.ds(0, T)])

    @pl.when(wid == NW - 2)
    def _():
        pltpu.make_async_copy(tbuf, tout_hbm.at[pl.ds(T, T)], tsem).wait()


def kernel(x_chunk, offset_t, hist_init, times_init):
    x_rows = x_chunk.reshape(T * B, D)
    off_vec = jnp.arange(16, dtype=jnp.float32) + jnp.asarray(offset_t, jnp.float32)
    return _sc_kernel(x_rows, off_vec, hist_init, times_init)


# speedup vs baseline: 1.7954x; 1.7954x over previous
import functools

import jax
import jax.numpy as jnp
from jax import lax
from jax.experimental import pallas as pl
from jax.experimental.pallas import tpu as pltpu
from jax.experimental.pallas import tpu_sc as plsc

DEPTH = 8192
D = 1024
T = 4096
B = 4

NC = 2
NS = 16
NW = NC * NS
ROWS = T // NW
CH = 8
NCHUNK = ROWS // CH
HCH = 16
NHR = ROWS // HCH

_MESH = plsc.VectorSubcoreMesh(core_axis_name="c", subcore_axis_name="s")


@functools.partial(
    pl.kernel,
    mesh=_MESH,
    out_type=(
        jax.ShapeDtypeStruct((DEPTH, D), jnp.float32),
        jax.ShapeDtypeStruct((DEPTH,), jnp.float32),
    ),
    scratch_types=[
        pltpu.VMEM((2, CH, B, D), jnp.float32),
        pltpu.VMEM((2, CH, D), jnp.float32),
        pltpu.VMEM_SHARED((NS, HCH, D), jnp.float32),
        pltpu.VMEM_SHARED((NS, 2, CH, D), jnp.float32),
        pltpu.VMEM((16,), jnp.float32),
        pltpu.VMEM((T // 4,), jnp.float32),
        pltpu.SemaphoreType.DMA,
        pltpu.SemaphoreType.DMA,
        pltpu.SemaphoreType.DMA,
        pltpu.SemaphoreType.DMA,
        pltpu.SemaphoreType.DMA,
        pltpu.SemaphoreType.DMA,
        pltpu.SemaphoreType.DMA,
        pltpu.SemaphoreType.DMA,
    ],
)
def _sc_kernel(x_hbm, off_hbm, hist_hbm, tin_hbm, out_hbm, tout_hbm,
               xbuf, obuf, hshared, oshared, offbuf, tbuf,
               xsem0, xsem1, osem0, osem1, odsem0, odsem1, hsem, tsem):
    cid = lax.axis_index("c")
    sid = lax.axis_index("s")
    wid = sid * NC + cid
    base = wid * ROWS
    xsems = (xsem0, xsem1)
    osems = (osem0, osem1)
    odsems = (odsem0, odsem1)

    def h_in(r):
        return pltpu.make_async_copy(
            hist_hbm.at[pl.ds(T + base + r * HCH, HCH)],
            hshared.at[sid], hsem)

    def h_out(r):
        return pltpu.make_async_copy(
            hshared.at[sid],
            out_hbm.at[pl.ds(base + r * HCH, HCH)], hsem)

    h_in(0).start()

    @pl.when(wid == NW - 1)
    def _():
        for k in range(4):
            pltpu.sync_copy(tin_hbm.at[pl.ds(T + k * (T // 4), T // 4)], tbuf)
            pltpu.sync_copy(tbuf, tout_hbm.at[pl.ds(k * (T // 4), T // 4)])

    @pl.when(wid == NW - 2)
    def _():
        pltpu.sync_copy(off_hbm, offbuf)
        offv = offbuf[...]
        for k in range(4):
            kbase = k * (T // 4)

            @plsc.parallel_loop(0, T // 64, 1, unroll=4)
            def _(j, kbase=kbase):
                tbuf[pl.ds(j * 16, 16)] = offv + lax.convert_element_type(
                    kbase + j * 16, jnp.float32)

            pltpu.sync_copy(tbuf, tout_hbm.at[pl.ds(T + kbase, T // 4)])

    def x_copy(c, b):
        return pltpu.make_async_copy(
            x_hbm.at[pl.ds(base + c * CH, CH)], xbuf.at[b], xsems[b])

    def o_stream(c):
        b = c & 1
        return pltpu.make_async_copy(obuf.at[b], oshared.at[sid, b], osems[b])

    def o_dma(c):
        b = c & 1
        return pltpu.make_async_copy(
            oshared.at[sid, b],
            out_hbm.at[pl.ds(T + base + c * CH, CH)], odsems[b])

    x_copy(0, 0).start()
    for c in range(NCHUNK):
        b = c & 1
        if c + 1 < NCHUNK:
            x_copy(c + 1, 1 - b).start()
        x_copy(c, b).wait()
        if c >= 1:
            o_stream(c - 1).wait()
            o_dma(c - 1).start()
        if c >= 2:
            o_dma(c - 2).wait()
        xb = xbuf.at[b]
        ob = obuf.at[b]

        @plsc.parallel_loop(0, CH * (D // 128), 1, unroll=2)
        def _(i, xb=xb, ob=ob):
            r = i // (D // 128)
            jt = i % (D // 128)
            for jw in range(8):
                d0 = jt * 128 + jw * 16
                a0 = xb[r, 0, pl.ds(d0, 16)]
                a1 = xb[r, 1, pl.ds(d0, 16)]
                a2 = xb[r, 2, pl.ds(d0, 16)]
                a3 = xb[r, 3, pl.ds(d0, 16)]
                ob[r, pl.ds(d0, 16)] = ((a0 + a1) + (a2 + a3)) * 0.25

        o_stream(c).start()

        r = c // 2
        if c % 2 == 0:
            h_in(r).wait()
            h_out(r).start()
        else:
            h_out(r).wait()
            if r + 1 < NHR:
                h_in(r + 1).start()

    o_stream(NCHUNK - 1).wait()
    o_dma(NCHUNK - 1).start()
    o_dma(NCHUNK - 2).wait()
    o_dma(NCHUNK - 1).wait()


def kernel(x_chunk, offset_t, hist_init, times_init):
    off_vec = jnp.arange(16, dtype=jnp.float32) + jnp.asarray(offset_t, jnp.float32)
    return _sc_kernel(x_chunk, off_vec, hist_init, times_init)


# speedup vs baseline: 1.8133x; 1.0100x over previous
import functools

import jax
import jax.numpy as jnp
from jax import lax
from jax.experimental import pallas as pl
from jax.experimental.pallas import tpu as pltpu
from jax.experimental.pallas import tpu_sc as plsc

DEPTH = 8192
D = 1024
T = 4096
B = 4

NC = 2
NS = 16
NW = NC * NS
ROWS = T // NW
CH = 8
NCHUNK = ROWS // CH
HCH = 16
NHR = ROWS // HCH

_MESH = plsc.VectorSubcoreMesh(core_axis_name="c", subcore_axis_name="s")


@functools.partial(
    pl.kernel,
    mesh=_MESH,
    out_type=(
        jax.ShapeDtypeStruct((DEPTH, D), jnp.float32),
        jax.ShapeDtypeStruct((DEPTH,), jnp.float32),
    ),
    scratch_types=[
        pltpu.VMEM((2, CH, B, D), jnp.float32),
        pltpu.VMEM((2, CH, D), jnp.float32),
        pltpu.VMEM_SHARED((NS, 2, HCH, D), jnp.float32),
        pltpu.VMEM((16,), jnp.float32),
        pltpu.VMEM((T,), jnp.float32),
        pltpu.SemaphoreType.DMA,
        pltpu.SemaphoreType.DMA,
        pltpu.SemaphoreType.DMA,
        pltpu.SemaphoreType.DMA,
        pltpu.SemaphoreType.DMA,
        pltpu.SemaphoreType.DMA,
        pltpu.SemaphoreType.DMA,
    ],
)
def _sc_kernel(x_hbm, off_hbm, hist_hbm, tin_hbm, out_hbm, tout_hbm,
               xbuf, obuf, hshared, offbuf, tbuf,
               xsem0, xsem1, osem0, osem1, hsem0, hsem1, tsem):
    cid = lax.axis_index("c")
    sid = lax.axis_index("s")
    wid = sid * NC + cid
    base = wid * ROWS
    xsems = (xsem0, xsem1)
    osems = (osem0, osem1)
    hsems = (hsem0, hsem1)

    def h_in(r):
        return pltpu.make_async_copy(
            hist_hbm.at[pl.ds(T + base + r * HCH, HCH)],
            hshared.at[sid, r % 2], hsems[r % 2])

    def h_out(r):
        return pltpu.make_async_copy(
            hshared.at[sid, r % 2],
            out_hbm.at[pl.ds(base + r * HCH, HCH)], hsems[r % 2])

    h_in(0).start()
    h_in(1).start()

    @pl.when(wid == NW - 1)
    def _():
        pltpu.async_copy(tin_hbm.at[pl.ds(T, T)], tbuf, tsem)

    @pl.when(wid == NW - 2)
    def _():
        pltpu.sync_copy(off_hbm, offbuf)
        offv = offbuf[...]

        @plsc.parallel_loop(0, T // 16, 1, unroll=4)
        def _(j):
            tbuf[pl.ds(j * 16, 16)] = offv + lax.convert_element_type(j * 16, jnp.float32)

        pltpu.async_copy(tbuf, tout_hbm.at[pl.ds(T, T)], tsem)

    def x_copy(c, b):
        return pltpu.make_async_copy(
            x_hbm.at[pl.ds(base + c * CH, CH)], xbuf.at[b], xsems[b])

    def o_copy(c, b):
        return pltpu.make_async_copy(
            obuf.at[b], out_hbm.at[pl.ds(T + base + c * CH, CH)], osems[b])

    x_copy(0, 0).start()
    for c in range(NCHUNK):
        b = c & 1
        if c + 1 < NCHUNK:
            x_copy(c + 1, 1 - b).start()
        x_copy(c, b).wait()
        if c >= 2:
            o_copy(c - 2, b).wait()
        xb = xbuf.at[b]
        ob = obuf.at[b]

        @plsc.parallel_loop(0, CH * (D // 128), 1, unroll=2)
        def _(i, xb=xb, ob=ob):
            r = i // (D // 128)
            jt = i % (D // 128)
            for jw in range(8):
                d0 = jt * 128 + jw * 16
                a0 = xb[r, 0, pl.ds(d0, 16)]
                a1 = xb[r, 1, pl.ds(d0, 16)]
                a2 = xb[r, 2, pl.ds(d0, 16)]
                a3 = xb[r, 3, pl.ds(d0, 16)]
                ob[r, pl.ds(d0, 16)] = ((a0 + a1) + (a2 + a3)) * 0.25

        o_copy(c, b).start()

        r = c // 2
        if c % 2 == 0:
            h_in(r).wait()
            h_out(r).start()
        else:
            h_out(r).wait()
            if r + 2 < NHR:
                h_in(r + 2).start()

    o_copy(NCHUNK - 2, 0).wait()
    o_copy(NCHUNK - 1, 1).wait()

    @pl.when(wid == NW - 1)
    def _():
        pltpu.make_async_copy(tin_hbm.at[pl.ds(T, T)], tbuf, tsem).wait()
        pltpu.sync_copy(tbuf, tout_hbm.at[pl.ds(0, T)])

    @pl.when(wid == NW - 2)
    def _():
        pltpu.make_async_copy(tbuf, tout_hbm.at[pl.ds(T, T)], tsem).wait()


def kernel(x_chunk, offset_t, hist_init, times_init):
    off_vec = jnp.arange(16, dtype=jnp.float32) + jnp.asarray(offset_t, jnp.float32)
    return _sc_kernel(x_chunk, off_vec, hist_init, times_init)
